# baseline (speedup 1.0000x reference)
import jax
import jax.numpy as jnp
from jax import lax
from jax.experimental import pallas as pl
from jax.experimental.pallas import tpu as pltpu
from jax.experimental.pallas import tpu_sc as plsc

B = 4096
L = 200
D = 64
V = 9999
N = B * L
NC = 2
NS = 16
NW = NC * NS
BT = 128
TC = 2
CB = BT * TC
BTP = BT + 1
TB = B // BT
N_CHUNKS_ALL = N // CB
PER_W = N_CHUNKS_ALL // NW
LANES = 16
VS = 625


def _body(year_hbm, month_hbm, pos_hbm, ix_hbm, out_hbm,
          ix0, g00, g10, g20,
          ix1, g01, g11, g21,
          ob, gsem0, gsem1, osem):
    cid = lax.axis_index("c")
    sid = lax.axis_index("s")
    wid = sid * NC + cid
    tbase = wid * PER_W

    ix_v = (ix0, ix1)
    g0 = (g00, g01)
    g1 = (g10, g11)
    g2 = (g20, g21)
    gsem = (gsem0, gsem1)

    lane = lax.iota(jnp.int32, LANES)
    i0 = []
    i1 = []
    fu = []
    for g in range(D // LANES):
        dd = g * LANES + lane
        i0.append(lax.shift_right_logical(dd, 3))
        i1.append(lax.bitwise_and(dd, 7))
    for u in range(TC):
        fu.append(jnp.full((LANES,), u, jnp.int32))

    def stage(t, s):
        off = pl.multiple_of(t * CB, CB)
        pltpu.sync_copy(ix_hbm.at[:, pl.ds(off, CB)], ix_v[s])
        pltpu.async_copy(year_hbm.at[ix_v[s].at[0]], g0[s], gsem[s])
        pltpu.async_copy(month_hbm.at[ix_v[s].at[1]], g1[s], gsem[s])
        pltpu.async_copy(pos_hbm.at[ix_v[s].at[2]], g2[s], gsem[s])

    def wait_gathers(s):
        pltpu.make_async_copy(year_hbm.at[ix_v[s].at[0]], g0[s], gsem[s]).wait()
        pltpu.make_async_copy(month_hbm.at[ix_v[s].at[1]], g1[s], gsem[s]).wait()
        pltpu.make_async_copy(pos_hbm.at[ix_v[s].at[2]], g2[s], gsem[s]).wait()

    def compute_fire(s, t):
        l = t // (TB // TC)
        tb = lax.rem(t, TB // TC) * TC

        def sub_body(u):
            def row_body(r4, rcarry):
                for uu in range(8):
                    r = r4 * 8 + uu
                    row = u * BT + r
                    i2 = jnp.full((LANES,), r, jnp.int32)
                    for g in range(D // LANES):
                        sl = pl.ds(g * LANES, LANES)
                        v = (g0[s][row, sl] + g1[s][row, sl]
                             + g2[s][row, sl])
                        plsc.store_scatter(ob, [i0[g], fu[u], i1[g], i2], v)
                return rcarry

            lax.fori_loop(0, BT // 8, row_body, 0, unroll=False)

        for u in range(TC):
            sub_body(u)
        pltpu.async_copy(ob.at[:, :, :, pl.ds(0, BT)],
                         out_hbm.at[l, :, pl.ds(tb, TC)], osem)

    def wait_out():
        pltpu.make_async_copy(ob.at[:, :, :, pl.ds(0, BT)],
                              out_hbm.at[0, :, pl.ds(0, TC)], osem).wait()

    stage(tbase, 0)
    stage(tbase + 1, 1)

    def pair_body(i2_, carry):
        t0 = tbase + 2 * i2_
        wait_gathers(0)
        compute_fire(0, t0)
        stage(t0 + 2, 0)
        wait_gathers(1)
        wait_out()
        compute_fire(1, t0 + 1)
        stage(t0 + 3, 1)
        wait_out()
        return carry

    lax.fori_loop(0, PER_W // 2 - 1, pair_body, 0, unroll=False)

    t0 = tbase + PER_W - 2
    wait_gathers(0)
    compute_fire(0, t0)
    wait_gathers(1)
    wait_out()
    compute_fire(1, t0 + 1)
    wait_out()


@jax.jit
def _temporal_embedding(year_embed, month_embed, pos_embed, ix):
    run = pl.kernel(
        _body,
        out_type=jax.ShapeDtypeStruct((L, D // 8, TB, 8, BT), jnp.float32),
        mesh=plsc.VectorSubcoreMesh(core_axis_name="c", subcore_axis_name="s"),
        scratch_types=[
            pltpu.VMEM((3, CB), jnp.int32),
            pltpu.VMEM((CB, D), jnp.float32),
            pltpu.VMEM((CB, D), jnp.float32),
            pltpu.VMEM((CB, D), jnp.float32),
            pltpu.VMEM((3, CB), jnp.int32),
            pltpu.VMEM((CB, D), jnp.float32),
            pltpu.VMEM((CB, D), jnp.float32),
            pltpu.VMEM((CB, D), jnp.float32),
            pltpu.VMEM((D // 8, TC, 8, BTP), jnp.float32),
            pltpu.SemaphoreType.DMA,
            pltpu.SemaphoreType.DMA,
            pltpu.SemaphoreType.DMA,
        ],
        compiler_params=pltpu.CompilerParams(use_tc_tiling_on_sc=False,
                                             needs_layout_passes=False),
    )
    return run(year_embed, month_embed, pos_embed, ix)


def kernel(x, absolute_position_embed, year_embed, month_embed):
    ix = x.astype(jnp.int32).transpose(2, 1, 0).reshape(3, N)
    out5 = _temporal_embedding(year_embed, month_embed,
                               absolute_position_embed, ix)
    return out5.transpose(2, 4, 0, 1, 3).reshape(B, L, D)


# speedup vs baseline: 1.0381x; 1.0381x over previous
import jax
import jax.numpy as jnp
from jax import lax
from jax.experimental import pallas as pl
from jax.experimental.pallas import tpu as pltpu
from jax.experimental.pallas import tpu_sc as plsc

B = 4096
L = 200
D = 64
V = 9999
N = B * L
NC = 2
NS = 16
NW = NC * NS
BT = 128
TC = 2
CB = BT * TC
BTP = BT + 1
TB = B // BT
N_CHUNKS_ALL = N // CB
PER_W = N_CHUNKS_ALL // NW
LANES = 16
VS = 625


def _body(year_hbm, month_hbm, pos_hbm, ix_hbm, out_hbm,
          ix0, g00, g10, g20,
          ix1, g01, g11, g21,
          ob, gsem0, gsem1, osem):
    cid = lax.axis_index("c")
    sid = lax.axis_index("s")
    wid = sid * NC + cid
    tbase = wid * PER_W

    ix_v = (ix0, ix1)
    g0 = (g00, g01)
    g1 = (g10, g11)
    g2 = (g20, g21)
    gsem = (gsem0, gsem1)

    lane = lax.iota(jnp.int32, LANES)
    i0 = []
    i1 = []
    fu = []
    for g in range(D // LANES):
        dd = g * LANES + lane
        i0.append(lax.shift_right_logical(dd, 3))
        i1.append(lax.bitwise_and(dd, 7))
    for u in range(TC):
        fu.append(jnp.full((LANES,), u, jnp.int32))

    def stage(t, s):
        off = pl.multiple_of(t * CB, CB)
        pltpu.sync_copy(ix_hbm.at[:, pl.ds(off, CB)], ix_v[s])
        pltpu.async_copy(year_hbm.at[ix_v[s].at[0]], g0[s], gsem[s])
        pltpu.async_copy(month_hbm.at[ix_v[s].at[1]], g1[s], gsem[s])
        pltpu.async_copy(pos_hbm.at[ix_v[s].at[2]], g2[s], gsem[s])

    def wait_gathers(s):
        pltpu.make_async_copy(year_hbm.at[ix_v[s].at[0]], g0[s], gsem[s]).wait()
        pltpu.make_async_copy(month_hbm.at[ix_v[s].at[1]], g1[s], gsem[s]).wait()
        pltpu.make_async_copy(pos_hbm.at[ix_v[s].at[2]], g2[s], gsem[s]).wait()

    def compute_fire(s, t):
        l = t // (TB // TC)
        tb = lax.rem(t, TB // TC) * TC

        def sub_body(u):
            def row_body(r4, rcarry):
                for uu in range(8):
                    r = r4 * 8 + uu
                    row = u * BT + r
                    i2 = jnp.full((LANES,), r, jnp.int32)
                    for g in range(D // LANES):
                        sl = pl.ds(g * LANES, LANES)
                        v = (g0[s][row, sl] + g1[s][row, sl]
                             + g2[s][row, sl])
                        plsc.store_scatter(ob, [fu[u], i0[g], i1[g], i2], v)
                return rcarry

            lax.fori_loop(0, BT // 8, row_body, 0, unroll=False)

        for u in range(TC):
            sub_body(u)
            pltpu.async_copy(ob.at[u, :, :, pl.ds(0, BT)],
                             out_hbm.at[l, :, tb + u], osem)

    def wait_out():
        for u in range(TC):
            pltpu.make_async_copy(ob.at[u, :, :, pl.ds(0, BT)],
                                  out_hbm.at[0, :, u], osem).wait()

    stage(tbase, 0)
    stage(tbase + 1, 1)

    def pair_body(i2_, carry):
        t0 = tbase + 2 * i2_
        wait_gathers(0)
        compute_fire(0, t0)
        stage(t0 + 2, 0)
        wait_gathers(1)
        wait_out()
        compute_fire(1, t0 + 1)
        stage(t0 + 3, 1)
        wait_out()
        return carry

    lax.fori_loop(0, PER_W // 2 - 1, pair_body, 0, unroll=False)

    t0 = tbase + PER_W - 2
    wait_gathers(0)
    compute_fire(0, t0)
    wait_gathers(1)
    wait_out()
    compute_fire(1, t0 + 1)
    wait_out()


@jax.jit
def _temporal_embedding(year_embed, month_embed, pos_embed, ix):
    run = pl.kernel(
        _body,
        out_type=jax.ShapeDtypeStruct((L, D // 8, TB, 8, BT), jnp.float32),
        mesh=plsc.VectorSubcoreMesh(core_axis_name="c", subcore_axis_name="s"),
        scratch_types=[
            pltpu.VMEM((3, CB), jnp.int32),
            pltpu.VMEM((CB, D), jnp.float32),
            pltpu.VMEM((CB, D), jnp.float32),
            pltpu.VMEM((CB, D), jnp.float32),
            pltpu.VMEM((3, CB), jnp.int32),
            pltpu.VMEM((CB, D), jnp.float32),
            pltpu.VMEM((CB, D), jnp.float32),
            pltpu.VMEM((CB, D), jnp.float32),
            pltpu.VMEM((TC, D // 8, 8, BTP), jnp.float32),
            pltpu.SemaphoreType.DMA,
            pltpu.SemaphoreType.DMA,
            pltpu.SemaphoreType.DMA,
        ],
        compiler_params=pltpu.CompilerParams(use_tc_tiling_on_sc=False,
                                             needs_layout_passes=False),
    )
    return run(year_embed, month_embed, pos_embed, ix)


def kernel(x, absolute_position_embed, year_embed, month_embed):
    ix = x.astype(jnp.int32).transpose(2, 1, 0).reshape(3, N)
    out5 = _temporal_embedding(year_embed, month_embed,
                               absolute_position_embed, ix)
    return out5.transpose(2, 4, 0, 1, 3).reshape(B, L, D)
